# Initial kernel scaffold; baseline (speedup 1.0000x reference)
import jax
import jax.numpy as jnp
from jax.experimental import pallas as pl


def kernel(x, c1_w, c1_b, gru_wih, gru_whh, gru_bih, gru_bhh, wq_w, wq_b, wk_w, wk_b, gcn_w, gcn_b, conv_w0, conv_b0, conv_w1, conv_b1, conv_w2, conv_b2, lin_w, lin_b):
    raise NotImplementedError("write your pallas kernel here")


# speedup vs baseline: 352.4446x; 352.4446x over previous
import math

import jax
import jax.numpy as jnp
from jax.experimental import pallas as pl
from jax.experimental.pallas import tpu as pltpu

B = 32
T = 12
N = 100
D = 64
GRU = 64
QK = 32
HOR = 3
NB = 3
TD = T * D
BN = B * N
NH = N * HOR
KTOT = T * N * D


def _dense_kernel(xTT_ref, wih_ref, whh_ref, bih_ref, bhh_ref,
                  wqT_ref, wqb_ref, wkT_ref, wkb_ref,
                  c1w_ref, c1b_ref, gcnwT_ref, gcnb_ref,
                  wcat0_ref, wcat1_ref, wcat2_ref, cb_ref,
                  hout_ref, bufA, bufB, at_ref):
    f32 = jnp.float32
    wih = wih_ref[...]
    bih = bih_ref[...]
    whh = whh_ref[...]
    bhh = bhh_ref[...]
    h = jnp.zeros((BN, GRU), f32)
    for t in range(T):
        xt = xTT_ref[:, t:t + 1]
        gi = xt * wih + bih
        gh = jax.lax.dot_general(h, whh, (((1,), (1,)), ((), ())),
                                 preferred_element_type=f32) + bhh
        r = jax.nn.sigmoid(gi[:, :GRU] + gh[:, :GRU])
        z = jax.nn.sigmoid(gi[:, GRU:2 * GRU] + gh[:, GRU:2 * GRU])
        n = jnp.tanh(gi[:, 2 * GRU:] + r * gh[:, 2 * GRU:])
        h = (1.0 - z) * n + z * h

    q = jnp.dot(h, wqT_ref[...], preferred_element_type=f32) + wqb_ref[...]
    k = jnp.dot(h, wkT_ref[...], preferred_element_type=f32) + wkb_ref[...]
    scale = 1.0 / math.sqrt(QK)
    for g in range(B):
        qg = q[g * N:(g + 1) * N, :]
        kg = k[g * N:(g + 1) * N, :]
        s = jax.lax.dot_general(qg, kg, (((1,), (1,)), ((), ())),
                                preferred_element_type=f32) * scale
        s = s - jnp.max(s, axis=1, keepdims=True)
        e = jnp.exp(s)
        w = e / jnp.sum(e, axis=1, keepdims=True)
        deg = jnp.sum(w, axis=0, keepdims=True)
        dis = jnp.where(deg > 0.0,
                        jax.lax.rsqrt(jnp.where(deg > 0.0, deg, 1.0)), 0.0)
        wn = w * dis
        at_ref[g] = wn.T * dis

    c1w = c1w_ref[...]
    c1b = c1b_ref[...]
    for t in range(T):
        xt = xTT_ref[:, t:t + 1]
        bufA[:, t * D:(t + 1) * D] = xt * c1w + c1b

    wcat_refs = (wcat0_ref, wcat1_ref, wcat2_ref)
    ksizes = (3, 5, 7)
    cur, other = bufA, bufB
    for b in range(NB):
        for t in range(T):
            wt = gcnwT_ref[b, t]
            other[:, t * D:(t + 1) * D] = jnp.dot(
                cur[:, t * D:(t + 1) * D], wt, preferred_element_type=f32)
        brow = gcnb_ref[b]
        for g in range(B):
            cur[g * N:(g + 1) * N, :] = jnp.dot(
                at_ref[g], other[g * N:(g + 1) * N, :],
                preferred_element_type=f32) + brow
        ksz = ksizes[b]
        p = ksz // 2
        wcat = wcat_refs[b][...]
        cbrow = cb_ref[b]
        dst = hout_ref if b == NB - 1 else other
        for t in range(T):
            lo = max(0, t - p)
            hi = min(T, t + p + 1)
            win = cur[:, lo * D:hi * D]
            wsl = wcat[(lo - (t - p)) * D:(hi - (t - p)) * D, :]
            o = jnp.dot(win, wsl, preferred_element_type=f32) + cbrow
            dst[:, t * D:(t + 1) * D] = jnp.where(o >= 0.0, o, 0.01 * o)
        cur, other = dst, cur


def _lin_kernel(x_ref, w_ref, b_ref, o_ref):
    i = pl.program_id(0)
    part = jax.lax.dot_general(x_ref[...], w_ref[...],
                               (((1,), (1,)), ((), ())),
                               preferred_element_type=jnp.float32)

    @pl.when(i == 0)
    def _init():
        o_ref[...] = part + b_ref[...]

    @pl.when(i > 0)
    def _acc():
        o_ref[...] += part


def kernel(x, c1_w, c1_b, gru_wih, gru_whh, gru_bih, gru_bhh,
           wq_w, wq_b, wk_w, wk_b, gcn_w, gcn_b,
           conv_w0, conv_b0, conv_w1, conv_b1, conv_w2, conv_b2,
           lin_w, lin_b):
    f32 = jnp.float32
    xTT = jnp.transpose(x, (0, 2, 1)).reshape(BN, T)
    wihT = gru_wih.reshape(1, 3 * GRU)
    bih2 = gru_bih.reshape(1, 3 * GRU)
    bhh2 = gru_bhh.reshape(1, 3 * GRU)
    wqT = wq_w.T
    wkT = wk_w.T
    wqb2 = wq_b.reshape(1, QK)
    wkb2 = wk_b.reshape(1, QK)
    c1w2 = c1_w.reshape(1, D)
    c1b2 = c1_b.reshape(1, D)
    gcn_wT = jnp.swapaxes(gcn_w, 2, 3)
    gcnb_flat = gcn_b.reshape(NB, 1, TD)
    wcat0 = jnp.transpose(conv_w0, (2, 1, 0)).reshape(3 * D, D)
    wcat1 = jnp.transpose(conv_w1, (2, 1, 0)).reshape(5 * D, D)
    wcat2 = jnp.transpose(conv_w2, (2, 1, 0)).reshape(7 * D, D)
    cbstack = jnp.stack([conv_b0, conv_b1, conv_b2]).reshape(NB, 1, D)

    hfull = pl.pallas_call(
        _dense_kernel,
        out_shape=jax.ShapeDtypeStruct((BN, TD), f32),
        scratch_shapes=[
            pltpu.VMEM((BN, TD), f32),
            pltpu.VMEM((BN, TD), f32),
            pltpu.VMEM((B, N, N), f32),
        ],
    )(xTT, wihT, gru_whh, bih2, bhh2, wqT, wqb2, wkT, wkb2,
      c1w2, c1b2, gcn_wT, gcnb_flat, wcat0, wcat1, wcat2, cbstack)

    xout = hfull.reshape(B, KTOT)
    nk = 12
    kc = KTOT // nk
    out = pl.pallas_call(
        _lin_kernel,
        grid=(nk,),
        in_specs=[
            pl.BlockSpec((B, kc), lambda i: (0, i)),
            pl.BlockSpec((NH, kc), lambda i: (0, i)),
            pl.BlockSpec((1, NH), lambda i: (0, 0)),
        ],
        out_specs=pl.BlockSpec((B, NH), lambda i: (0, 0)),
        out_shape=jax.ShapeDtypeStruct((B, NH), f32),
    )(xout, lin_w, lin_b.reshape(1, NH))
    return out
